# baseline (device time: 140462 ns/iter reference)
import jax
import jax.numpy as jnp
from jax import lax
from jax.experimental import pallas as pl
from jax.experimental.pallas import tpu as pltpu

N_DEV = 4
M_BLK = 1024
K_BLK = 1024
N_OUT = 8192
NT = 1024
N_TILES = N_OUT // NT


def kernel(x, w_mat):
    k_total = x.shape[0]
    assert x.shape == (k_total, K_BLK)
    assert w_mat.shape == (k_total, N_OUT)
    x_bf16 = x.astype(jnp.bfloat16)

    def body(x_ref, xb_ref, w_ref, out_ref, yacc, xfull, xloc, af32, wt,
             abuf, send_sems, recv_sems, asend, arecv, wsem, lsem):
        me = lax.axis_index("i")

        local_cp = pltpu.make_async_copy(
            x_ref.at[pl.ds(me * M_BLK, M_BLK), :], xloc, lsem)
        local_cp.start()

        bsem = pltpu.get_barrier_semaphore()
        for d in (1, 2, 3):
            pl.semaphore_signal(
                bsem, inc=1,
                device_id=((me + d) % N_DEV,),
                device_id_type=pl.DeviceIdType.MESH)
        pl.semaphore_wait(bsem, N_DEV - 1)

        def make_rdma(d):
            t = (me + d) % N_DEV
            return pltpu.make_async_remote_copy(
                src_ref=xb_ref.at[pl.ds(t * M_BLK, M_BLK), :],
                dst_ref=xfull.at[d],
                send_sem=send_sems.at[d],
                recv_sem=recv_sems.at[d],
                device_id=(t,),
                device_id_type=pl.DeviceIdType.MESH)

        rdmas = {d: make_rdma(d) for d in (1, 2, 3)}
        rdmas[1].start()
        rdmas[3].start()

        order = (0, 1, 3, 2)
        steps = [(d, nt) for d in order for nt in range(N_TILES)]

        def start_w_dma(idx):
            d, nt = steps[idx]
            j = (me + (N_DEV - d)) % N_DEV
            cp = pltpu.make_async_copy(
                w_ref.at[pl.ds(j * K_BLK, K_BLK), pl.ds(nt * NT, NT)],
                wt.at[idx % 2],
                wsem.at[idx % 2])
            cp.start()
            return cp

        cps = {0: start_w_dma(0)}
        local_cp.wait()
        maxparts = []
        for idx, (d, nt) in enumerate(steps):
            if idx + 1 < len(steps):
                cps[idx + 1] = start_w_dma(idx + 1)
            if nt == 0 and d == 1:
                rdmas[1].wait_send()
                rdmas[3].wait_send()
                rdmas[2].start()
            if nt == 0 and d != 0:
                rdmas[d].wait_recv()
                af32[...] = xfull[d].astype(jnp.float32)
            cps[idx].wait()
            a = xloc if d == 0 else af32
            acc = jnp.dot(a[...], wt[idx % 2],
                          preferred_element_type=jnp.float32)
            sl = slice(nt * NT, (nt + 1) * NT)
            if d == order[0]:
                yacc[:, sl] = acc
            else:
                val = yacc[:, sl] + acc
                yacc[:, sl] = val
                if d == order[-1]:
                    maxparts.append(jnp.max(jnp.abs(val)))

        loc = jnp.max(jnp.stack(maxparts))
        abuf[0] = jnp.full((8, 128), loc, jnp.float32)
        ars = []
        for d in (1, 2, 3):
            t = (me + d) % N_DEV
            ar = pltpu.make_async_remote_copy(
                src_ref=abuf.at[0],
                dst_ref=abuf.at[d],
                send_sem=asend.at[d],
                recv_sem=arecv.at[d],
                device_id=(t,),
                device_id_type=pl.DeviceIdType.MESH)
            ar.start()
            ars.append(ar)
        rdmas[2].wait_send()
        for ar in ars:
            ar.wait()
        amax = jnp.max(abuf[...])

        scale = amax / 127.0
        inv = 127.0 / amax
        stores = {}
        for i in range(N_TILES):
            sl = slice(i * NT, (i + 1) * NT)
            q = jnp.clip(jnp.round(yacc[:, sl] * inv), -127.0, 127.0)
            yacc[:, sl] = q * scale
            if i >= 2:
                stores[i - 2].wait()
            cp = pltpu.make_async_copy(
                yacc.at[:, sl], out_ref.at[:, sl], wsem.at[i % 2])
            cp.start()
            stores[i] = cp
        stores[N_TILES - 2].wait()
        stores[N_TILES - 1].wait()

    return pl.pallas_call(
        body,
        out_shape=jax.ShapeDtypeStruct((M_BLK, N_OUT), jnp.float32),
        in_specs=[
            pl.BlockSpec(memory_space=pl.ANY),
            pl.BlockSpec(memory_space=pl.ANY),
            pl.BlockSpec(memory_space=pl.ANY),
        ],
        out_specs=pl.BlockSpec(memory_space=pl.ANY),
        scratch_shapes=[
            pltpu.VMEM((M_BLK, N_OUT), jnp.float32),
            pltpu.VMEM((N_DEV, M_BLK, K_BLK), jnp.bfloat16),
            pltpu.VMEM((M_BLK, K_BLK), jnp.float32),
            pltpu.VMEM((M_BLK, K_BLK), jnp.float32),
            pltpu.VMEM((2, K_BLK, NT), jnp.float32),
            pltpu.VMEM((N_DEV, 8, 128), jnp.float32),
            pltpu.SemaphoreType.DMA((N_DEV,)),
            pltpu.SemaphoreType.DMA((N_DEV,)),
            pltpu.SemaphoreType.DMA((N_DEV,)),
            pltpu.SemaphoreType.DMA((N_DEV,)),
            pltpu.SemaphoreType.DMA((2,)),
            pltpu.SemaphoreType.DMA,
        ],
        compiler_params=pltpu.CompilerParams(
            collective_id=0,
            vmem_limit_bytes=64 * 1024 * 1024,
        ),
    )(x, x_bf16, w_mat)


# device time: 136313 ns/iter; 1.0304x vs baseline; 1.0304x over previous
import jax
import jax.numpy as jnp
from jax import lax
from jax.experimental import pallas as pl
from jax.experimental.pallas import tpu as pltpu

N_DEV = 4
M_BLK = 1024
K_BLK = 1024
N_OUT = 8192
NT = 1024
N_TILES = N_OUT // NT


def kernel(x, w_mat):
    k_total = x.shape[0]
    assert x.shape == (k_total, K_BLK)
    assert w_mat.shape == (k_total, N_OUT)
    x_bf16 = x.astype(jnp.bfloat16)

    def body(x_ref, xb_ref, w_ref, out_ref, yacc, xfull, xloc, af32, wt, abuf,
             send_sems, recv_sems, asend, arecv, wsem, lsem):
        me = lax.axis_index("i")

        bsem = pltpu.get_barrier_semaphore()
        for d in (1, 2, 3):
            pl.semaphore_signal(
                bsem, inc=1,
                device_id=((me + d) % N_DEV,),
                device_id_type=pl.DeviceIdType.MESH)
        pl.semaphore_wait(bsem, N_DEV - 1)

        def make_rdma(d):
            t = (me + d) % N_DEV
            return pltpu.make_async_remote_copy(
                src_ref=xb_ref.at[pl.ds(t * M_BLK, M_BLK), :],
                dst_ref=xfull.at[d],
                send_sem=send_sems.at[d],
                recv_sem=recv_sems.at[d],
                device_id=(t,),
                device_id_type=pl.DeviceIdType.MESH)

        rdmas = {d: make_rdma(d) for d in (1, 2, 3)}
        rdmas[1].start()
        rdmas[3].start()

        local_cp = pltpu.make_async_copy(
            x_ref.at[pl.ds(me * M_BLK, M_BLK), :], xloc, lsem)
        local_cp.start()

        order = (0, 1, 3, 2)
        steps = [(d, nt) for d in order for nt in range(N_TILES)]

        def start_w_dma(idx):
            d, nt = steps[idx]
            j = (me + (N_DEV - d)) % N_DEV
            cp = pltpu.make_async_copy(
                w_ref.at[pl.ds(j * K_BLK, K_BLK), pl.ds(nt * NT, NT)],
                wt.at[idx % 2],
                wsem.at[idx % 2])
            cp.start()
            return cp

        cps = {0: start_w_dma(0)}
        local_cp.wait()
        for idx, (d, nt) in enumerate(steps):
            if idx + 1 < len(steps):
                cps[idx + 1] = start_w_dma(idx + 1)
            if nt == 0 and d == 1:
                rdmas[1].wait_send()
                rdmas[3].wait_send()
                rdmas[2].start()
            if nt == 0 and d != 0:
                rdmas[d].wait_recv()
                af32[...] = xfull[d].astype(jnp.float32)
            cps[idx].wait()
            a = xloc if d == 0 else af32
            acc = jnp.dot(a[...], wt[idx % 2],
                          preferred_element_type=jnp.float32)
            sl = slice(nt * NT, (nt + 1) * NT)
            if d == order[0]:
                yacc[:, sl] = acc
            else:
                yacc[:, sl] = yacc[:, sl] + acc

        rdmas[2].wait_send()

        loc = jnp.max(jnp.stack(
            [jnp.max(jnp.abs(yacc[:, i * NT:(i + 1) * NT]))
             for i in range(N_TILES)]))
        abuf[0] = jnp.full((8, 128), loc, jnp.float32)
        ars = []
        for d in (1, 2, 3):
            t = (me + d) % N_DEV
            ar = pltpu.make_async_remote_copy(
                src_ref=abuf.at[0],
                dst_ref=abuf.at[d],
                send_sem=asend.at[d],
                recv_sem=arecv.at[d],
                device_id=(t,),
                device_id_type=pl.DeviceIdType.MESH)
            ar.start()
            ars.append(ar)
        for ar in ars:
            ar.wait()
        amax = jnp.max(abuf[...])

        scale = amax / 127.0
        inv = 127.0 / amax
        stores = {}
        for i in range(N_TILES):
            sl = slice(i * NT, (i + 1) * NT)
            q = jnp.clip(jnp.round(yacc[:, sl] * inv), -127.0, 127.0)
            yacc[:, sl] = q * scale
            if i >= 2:
                stores[i - 2].wait()
            cp = pltpu.make_async_copy(
                yacc.at[:, sl], out_ref.at[:, sl], wsem.at[i % 2])
            cp.start()
            stores[i] = cp
        stores[N_TILES - 2].wait()
        stores[N_TILES - 1].wait()

    return pl.pallas_call(
        body,
        out_shape=jax.ShapeDtypeStruct((M_BLK, N_OUT), jnp.float32),
        in_specs=[
            pl.BlockSpec(memory_space=pl.ANY),
            pl.BlockSpec(memory_space=pl.ANY),
            pl.BlockSpec(memory_space=pl.ANY),
        ],
        out_specs=pl.BlockSpec(memory_space=pl.ANY),
        scratch_shapes=[
            pltpu.VMEM((M_BLK, N_OUT), jnp.float32),
            pltpu.VMEM((N_DEV, M_BLK, K_BLK), jnp.bfloat16),
            pltpu.VMEM((M_BLK, K_BLK), jnp.float32),
            pltpu.VMEM((M_BLK, K_BLK), jnp.float32),
            pltpu.VMEM((2, K_BLK, NT), jnp.float32),
            pltpu.VMEM((N_DEV, 8, 128), jnp.float32),
            pltpu.SemaphoreType.DMA((N_DEV,)),
            pltpu.SemaphoreType.DMA((N_DEV,)),
            pltpu.SemaphoreType.DMA((N_DEV,)),
            pltpu.SemaphoreType.DMA((N_DEV,)),
            pltpu.SemaphoreType.DMA((2,)),
            pltpu.SemaphoreType.DMA,
        ],
        compiler_params=pltpu.CompilerParams(
            collective_id=0,
            vmem_limit_bytes=64 * 1024 * 1024,
        ),
    )(x, x_bf16, w_mat)
